# initial kernel scaffold (unmeasured)
import jax
import jax.numpy as jnp
from jax import lax
from jax.experimental import pallas as pl
from jax.experimental.pallas import tpu as pltpu


def kernel(x, dest):
    m_per, n = x.shape
    m_glob = 2 * m_per
    dest2d = dest.reshape(1, m_per)

    def body(x_ref, dest_ref, out_ref, xg, destg, send_sems, recv_sems):
        my_x = lax.axis_index("x")
        my_y = lax.axis_index("y")
        nbr = (my_x, 1 - my_y)

        barrier_sem = pltpu.get_barrier_semaphore()
        pl.semaphore_signal(
            barrier_sem, inc=1, device_id=nbr,
            device_id_type=pl.DeviceIdType.MESH,
        )
        pl.semaphore_wait(barrier_sem, 1)

        xg[pl.ds(my_y * m_per, m_per), :] = x_ref[...]
        destg[pl.ds(my_y, 1), :] = dest_ref[...]

        rdma_x = pltpu.make_async_remote_copy(
            src_ref=x_ref,
            dst_ref=xg.at[pl.ds(my_y * m_per, m_per)],
            send_sem=send_sems.at[0],
            recv_sem=recv_sems.at[0],
            device_id=nbr,
            device_id_type=pl.DeviceIdType.MESH,
        )
        rdma_d = pltpu.make_async_remote_copy(
            src_ref=dest_ref,
            dst_ref=destg.at[pl.ds(my_y, 1)],
            send_sem=send_sems.at[1],
            recv_sem=recv_sems.at[1],
            device_id=nbr,
            device_id_type=pl.DeviceIdType.MESH,
        )
        rdma_x.start()
        rdma_d.start()
        rdma_d.wait()
        rdma_x.wait()

        msk = destg[...] == my_y
        mflat = msk.reshape(1, m_glob)
        m_f32 = mflat.astype(jnp.float32)
        row_i = lax.broadcasted_iota(jnp.int32, (m_glob, m_glob), 0)
        col_i = lax.broadcasted_iota(jnp.int32, (m_glob, m_glob), 1)
        tri = (row_i < col_i).astype(jnp.float32)
        ex = jnp.dot(m_f32, tri, preferred_element_type=jnp.float32)
        r_iota = lax.broadcasted_iota(jnp.float32, (m_per, m_glob), 0)
        sel = jnp.where((ex == r_iota) & mflat, 1.0, 0.0)
        out_ref[...] = jnp.dot(sel, xg[...], preferred_element_type=jnp.float32)

    return pl.pallas_call(
        body,
        out_shape=jax.ShapeDtypeStruct((m_per, n), jnp.float32),
        in_specs=[
            pl.BlockSpec(memory_space=pltpu.VMEM),
            pl.BlockSpec(memory_space=pltpu.VMEM),
        ],
        out_specs=pl.BlockSpec(memory_space=pltpu.VMEM),
        scratch_shapes=[
            pltpu.VMEM((m_glob, n), jnp.float32),
            pltpu.VMEM((2, m_per), jnp.int32),
            pltpu.SemaphoreType.DMA((2,)),
            pltpu.SemaphoreType.DMA((2,)),
        ],
        compiler_params=pltpu.CompilerParams(collective_id=0),
    )(x, dest2d)


# baseline (device time: 12476 ns/iter reference)
import jax
import jax.numpy as jnp
from jax import lax
from jax.experimental import pallas as pl
from jax.experimental.pallas import tpu as pltpu


def kernel(x, dest):
    m_per, n = x.shape
    m_glob = 2 * m_per
    dest2d = dest.reshape(1, m_per)

    def body(x_ref, dest_ref, out_ref, xg, destg, send_sems, recv_sems):
        my_x = lax.axis_index("x")
        my_y = lax.axis_index("y")
        nbr = (my_x, 1 - my_y)

        barrier_sem = pltpu.get_barrier_semaphore()
        pl.semaphore_signal(
            barrier_sem, inc=1, device_id=nbr,
            device_id_type=pl.DeviceIdType.MESH,
        )
        pl.semaphore_wait(barrier_sem, 1)

        xg[pl.ds(my_y * m_per, m_per), :] = x_ref[...]
        destg[pl.ds(my_y, 1), :] = dest_ref[...]

        rdma_x = pltpu.make_async_remote_copy(
            src_ref=x_ref,
            dst_ref=xg.at[pl.ds(my_y * m_per, m_per)],
            send_sem=send_sems.at[0],
            recv_sem=recv_sems.at[0],
            device_id=nbr,
            device_id_type=pl.DeviceIdType.MESH,
        )
        rdma_d = pltpu.make_async_remote_copy(
            src_ref=dest_ref,
            dst_ref=destg.at[pl.ds(my_y, 1)],
            send_sem=send_sems.at[1],
            recv_sem=recv_sems.at[1],
            device_id=nbr,
            device_id_type=pl.DeviceIdType.MESH,
        )
        rdma_x.start()
        rdma_d.start()
        rdma_d.wait()
        rdma_x.wait()

        d_f = destg[...].astype(jnp.float32)
        y_f = my_y.astype(jnp.float32)
        diff = d_f - y_f
        m_f32 = (1.0 - diff * diff).reshape(1, m_glob)
        row_i = lax.broadcasted_iota(jnp.int32, (m_glob, m_glob), 0)
        col_i = lax.broadcasted_iota(jnp.int32, (m_glob, m_glob), 1)
        tri = (row_i < col_i).astype(jnp.float32)
        ex = jnp.dot(m_f32, tri, preferred_element_type=jnp.float32)
        exi = ex.astype(jnp.int32)
        r_iota = lax.broadcasted_iota(jnp.int32, (m_per, m_glob), 0)
        onehot = jnp.where(exi == r_iota, 1.0, 0.0)
        sel = onehot * m_f32
        out_ref[...] = jnp.dot(sel, xg[...], preferred_element_type=jnp.float32)

    return pl.pallas_call(
        body,
        out_shape=jax.ShapeDtypeStruct((m_per, n), jnp.float32),
        in_specs=[
            pl.BlockSpec(memory_space=pltpu.VMEM),
            pl.BlockSpec(memory_space=pltpu.VMEM),
        ],
        out_specs=pl.BlockSpec(memory_space=pltpu.VMEM),
        scratch_shapes=[
            pltpu.VMEM((m_glob, n), jnp.float32),
            pltpu.VMEM((2, m_per), jnp.int32),
            pltpu.SemaphoreType.DMA((2,)),
            pltpu.SemaphoreType.DMA((2,)),
        ],
        compiler_params=pltpu.CompilerParams(collective_id=0),
    )(x, dest2d)
